# baseline (speedup 1.0000x reference)
import functools

import jax
import jax.numpy as jnp
from jax import lax
from jax.experimental import pallas as pl
from jax.experimental.pallas import tpu as pltpu
from jax.experimental.pallas import tpu_sc as plsc

VOCAB = 100000
MAX_LEN = 200
EMB = 128
B = 1024
T = 200

NC = 2
NS = 16
NW = NC * NS

CHUNK = 40
CPB = T // CHUNK
BPW = B // NW
NCH = CPB * BPW

TOK_BLK = 2000
GRID = VOCAB // TOK_BLK


def _prep_body(tok_ref, pos_ref, m_ref, qpos_ref, acc_ref):
    i = pl.program_id(0)

    @pl.when(i == 0)
    def _():
        acc_ref[...] = jnp.zeros((1, 128), jnp.float32)

    acc_ref[...] += jnp.sum(tok_ref[...], axis=0, keepdims=True)

    @pl.when(i == GRID - 1)
    def _():
        m_tok = jnp.sum(acc_ref[...]) / float(VOCAB * EMB)
        m_ref[...] = jnp.full((8, 128), m_tok, jnp.float32)
        pos = pos_ref[...]
        m_pos = jnp.sum(pos) / float(MAX_LEN * EMB)
        qpos_ref[...] = (pos > m_pos).astype(jnp.float32)


def _prep(token_table, pos_table):
    return pl.pallas_call(
        _prep_body,
        grid=(GRID,),
        in_specs=[
            pl.BlockSpec((TOK_BLK, EMB), lambda i: (i, 0)),
            pl.BlockSpec((MAX_LEN, EMB), lambda i: (0, 0)),
        ],
        out_specs=[
            pl.BlockSpec((8, 128), lambda i: (0, 0)),
            pl.BlockSpec((MAX_LEN, EMB), lambda i: (0, 0)),
        ],
        out_shape=[
            jax.ShapeDtypeStruct((8, 128), jnp.float32),
            jax.ShapeDtypeStruct((MAX_LEN, EMB), jnp.float32),
        ],
        scratch_shapes=[pltpu.VMEM((1, 128), jnp.float32)],
    )(token_table, pos_table)


_mesh = plsc.VectorSubcoreMesh(
    core_axis_name="c", subcore_axis_name="s", num_cores=NC, num_subcores=NS
)


@functools.partial(
    pl.kernel,
    out_type=jax.ShapeDtypeStruct((B, T, EMB), jnp.float32),
    mesh=_mesh,
    scratch_types=[
        pltpu.VMEM((NCH, CHUNK), jnp.int32),
        pltpu.VMEM((CHUNK, EMB), jnp.float32),
        pltpu.VMEM((CHUNK, EMB), jnp.float32),
        pltpu.VMEM((CHUNK, EMB), jnp.float32),
        pltpu.VMEM((CHUNK, EMB), jnp.float32),
        pltpu.VMEM((MAX_LEN, EMB), jnp.float32),
        pltpu.VMEM((16,), jnp.float32),
        pltpu.SemaphoreType.DMA,
        pltpu.SemaphoreType.DMA,
        pltpu.SemaphoreType.DMA,
        pltpu.SemaphoreType.DMA,
    ],
)
def _sc_lookup(seq_hbm, tok_hbm, m_hbm, qpos_hbm, out_hbm,
               idx_all, g0, g1, o0, o1, qpos_v, m_v,
               gsem0, gsem1, osem0, osem1):
    wid = lax.axis_index("s") * NC + lax.axis_index("c")
    base_b = wid * BPW

    pltpu.sync_copy(seq_hbm.at[pl.ds(wid * NCH, NCH)], idx_all)
    pltpu.sync_copy(qpos_hbm, qpos_v)
    pltpu.sync_copy(m_hbm, m_v)
    vm = m_v[...]

    def gstart(i, g, gsem):
        pltpu.async_copy(tok_hbm.at[idx_all.at[i]], g, gsem)

    def gwait(g, gsem):
        pltpu.make_async_copy(
            tok_hbm.at[idx_all.at[0]], g, gsem).wait()

    def ostart(i, o, osem):
        off = pl.multiple_of(lax.rem(i, CPB) * CHUNK, 8)
        pltpu.async_copy(
            o, out_hbm.at[base_b + lax.div(i, CPB), pl.ds(off, CHUNK)],
            osem)

    def owait(o, osem):
        pltpu.make_async_copy(
            o, out_hbm.at[base_b, pl.ds(0, CHUNK)], osem).wait()

    def compute(i, g, o):
        off = lax.rem(i, CPB) * CHUNK

        def body_r(r, c):
            tr = off + r
            for j in range(EMB // 16):
                sl = pl.ds(j * 16, 16)
                v = g[r, sl]
                qp = qpos_v[tr, sl]
                o[r, sl] = jnp.where(v > vm, 1.0, 0.0) + qp
            return c

        lax.fori_loop(0, CHUNK, body_r, 0)

    def slot(i, g, o, gsem, osem, first, last):
        gwait(g, gsem)
        if not first:
            owait(o, osem)
        compute(i, g, o)
        ostart(i, o, osem)
        if not last:
            gstart(i + 2, g, gsem)

    gstart(0, g0, gsem0)
    gstart(1, g1, gsem1)
    slot(0, g0, o0, gsem0, osem0, True, False)
    slot(1, g1, o1, gsem1, osem1, True, False)

    def pair(k, carry):
        e = 2 * k
        slot(e, g0, o0, gsem0, osem0, False, False)
        slot(e + 1, g1, o1, gsem1, osem1, False, False)
        return carry

    lax.fori_loop(1, NCH // 2 - 1, pair, 0)
    slot(NCH - 2, g0, o0, gsem0, osem0, False, True)
    slot(NCH - 1, g1, o1, gsem1, osem1, False, True)
    owait(o0, osem0)
    owait(o1, osem1)


def kernel(seq, token_table, pos_table):
    m8, qpos = _prep(token_table, pos_table)
    mvec = m8[0, :16]
    seq2 = seq.astype(jnp.int32).reshape(B * T // CHUNK, CHUNK)
    return _sc_lookup(seq2, token_table, mvec, qpos)


# speedup vs baseline: 2.2500x; 2.2500x over previous
import functools

import jax
import jax.numpy as jnp
from jax import lax
from jax.experimental import pallas as pl
from jax.experimental.pallas import tpu as pltpu
from jax.experimental.pallas import tpu_sc as plsc

VOCAB = 100000
MAX_LEN = 200
EMB = 128
B = 1024
T = 200

NC = 2
NS = 16
NW = NC * NS

BPW = B // NW
GS0 = 104
GS1 = T - GS0

TOK_BLK = 2000
GRID = VOCAB // TOK_BLK


def _prep_body(tok_ref, pos_ref, m_ref, qpos_ref, acc_ref):
    i = pl.program_id(0)

    @pl.when(i == 0)
    def _():
        acc_ref[...] = jnp.zeros((1, 128), jnp.float32)

    acc_ref[...] += jnp.sum(tok_ref[...], axis=0, keepdims=True)

    @pl.when(i == GRID - 1)
    def _():
        m_tok = jnp.sum(acc_ref[...]) / float(VOCAB * EMB)
        m_ref[...] = jnp.full((8, 128), m_tok, jnp.float32)
        pos = pos_ref[...]
        m_pos = jnp.sum(pos) / float(MAX_LEN * EMB)
        qpos_ref[...] = (pos > m_pos).astype(jnp.float32)


def _prep(token_table, pos_table):
    return pl.pallas_call(
        _prep_body,
        grid=(GRID,),
        in_specs=[
            pl.BlockSpec((TOK_BLK, EMB), lambda i: (i, 0)),
            pl.BlockSpec((MAX_LEN, EMB), lambda i: (0, 0)),
        ],
        out_specs=[
            pl.BlockSpec((8, 128), lambda i: (0, 0)),
            pl.BlockSpec((MAX_LEN, EMB), lambda i: (0, 0)),
        ],
        out_shape=[
            jax.ShapeDtypeStruct((8, 128), jnp.float32),
            jax.ShapeDtypeStruct((MAX_LEN, EMB), jnp.float32),
        ],
        scratch_shapes=[pltpu.VMEM((1, 128), jnp.float32)],
    )(token_table, pos_table)


_mesh = plsc.VectorSubcoreMesh(
    core_axis_name="c", subcore_axis_name="s", num_cores=NC, num_subcores=NS
)


@functools.partial(
    pl.kernel,
    out_type=jax.ShapeDtypeStruct((B, T, EMB), jnp.float32),
    mesh=_mesh,
    scratch_types=[
        pltpu.VMEM((2 * BPW, GS0), jnp.int32),
        pltpu.VMEM((T, EMB), jnp.float32),
        pltpu.VMEM((T, EMB), jnp.float32),
        pltpu.VMEM((T, EMB), jnp.float32),
        pltpu.VMEM((MAX_LEN, EMB), jnp.float32),
        pltpu.VMEM((16,), jnp.float32),
        pltpu.SemaphoreType.DMA,
        pltpu.SemaphoreType.DMA,
        pltpu.SemaphoreType.DMA,
        pltpu.SemaphoreType.DMA,
        pltpu.SemaphoreType.DMA,
        pltpu.SemaphoreType.DMA,
    ],
)
def _sc_lookup(seq_hbm, tok_hbm, m_hbm, qpos_hbm, out_hbm,
               idx_all, g0, g1, g2, qpos_v, m_v,
               gsem0, gsem1, gsem2, osem0, osem1, osem2):
    wid = lax.axis_index("s") * NC + lax.axis_index("c")
    base_b = wid * BPW

    pltpu.sync_copy(seq_hbm.at[pl.ds(2 * base_b, 2 * BPW)], idx_all)
    pltpu.sync_copy(qpos_hbm, qpos_v)
    pltpu.sync_copy(m_hbm, m_v)
    vm = m_v[...]

    def gstart(i, g, gsem):
        pltpu.async_copy(
            tok_hbm.at[idx_all.at[2 * i]], g.at[pl.ds(0, GS0)], gsem)
        pltpu.async_copy(
            tok_hbm.at[idx_all.at[2 * i + 1]], g.at[pl.ds(GS1, GS0)], gsem)

    def gwait(g, gsem):
        pltpu.make_async_copy(
            tok_hbm.at[idx_all.at[0]], g.at[pl.ds(0, GS0)], gsem).wait()
        pltpu.make_async_copy(
            tok_hbm.at[idx_all.at[0]], g.at[pl.ds(GS1, GS0)], gsem).wait()

    def ostart(i, g, osem):
        pltpu.async_copy(g, out_hbm.at[base_b + i], osem)

    def owait(g, osem):
        pltpu.make_async_copy(g, out_hbm.at[base_b], osem).wait()

    def compute(g):
        def body_r(r, c):
            for j in range(EMB // 16):
                sl = pl.ds(j * 16, 16)
                v = g[r, sl]
                qp = qpos_v[r, sl]
                g[r, sl] = jnp.where(v > vm, 1.0, 0.0) + qp
            return c

        lax.fori_loop(0, T, body_r, 0)

    slots = ((g0, gsem0, osem0), (g1, gsem1, osem1), (g2, gsem2, osem2))

    def step(i, cur, nxt, prefetch, drain):
        g, gsem, osem = cur
        if prefetch:
            gn, gsemn, osemn = nxt
            if drain:
                owait(gn, osemn)
            gstart(i + 1, gn, gsemn)
        gwait(g, gsem)
        compute(g)
        ostart(i, g, osem)

    gstart(0, g0, gsem0)
    step(0, slots[0], slots[1], True, False)
    step(1, slots[1], slots[2], True, False)

    def group(k, carry):
        c = 3 * k + 2
        step(c + 0, slots[2], slots[0], True, True)
        step(c + 1, slots[0], slots[1], True, True)
        step(c + 2, slots[1], slots[2], True, True)
        return carry

    lax.fori_loop(0, (BPW - 5) // 3, group, 0)
    step(BPW - 3, slots[2], slots[0], True, True)
    step(BPW - 2, slots[0], slots[1], True, True)
    step(BPW - 1, slots[1], slots[2], False, False)
    owait(g2, osem2)
    owait(g0, osem0)
    owait(g1, osem1)


def kernel(seq, token_table, pos_table):
    m8, qpos = _prep(token_table, pos_table)
    mvec = m8[0, :16]
    s = seq.astype(jnp.int32)
    seq3 = jnp.stack([s[:, :GS0], s[:, GS1:]], axis=1).reshape(2 * B, GS0)
    return _sc_lookup(seq3, token_table, mvec, qpos)


# speedup vs baseline: 2.5417x; 1.1297x over previous
import functools

import jax
import jax.numpy as jnp
from jax import lax
from jax.experimental import pallas as pl
from jax.experimental.pallas import tpu as pltpu
from jax.experimental.pallas import tpu_sc as plsc

VOCAB = 100000
MAX_LEN = 200
EMB = 128
B = 1024
T = 200

NC = 2
NS = 16
NW = NC * NS

BPW = B // NW
GS0 = 104
GS1 = T - GS0

TOK_BLK = 5000
GRID = VOCAB // TOK_BLK
SUB = TOK_BLK // 8

_BF16_ONE = 0x3F80


def _prep_body(tok_ref, pos_ref, m_ref, qpos_ref, acc_ref):
    i = pl.program_id(0)

    @pl.when(i == 0)
    def _():
        acc_ref[...] = jnp.zeros((8, 128), jnp.float32)

    acc_ref[...] += jnp.sum(
        tok_ref[...].reshape(SUB, 8, 128), axis=0)

    @pl.when(i == GRID - 1)
    def _():
        m_tok = jnp.sum(acc_ref[...]) / float(VOCAB * EMB)
        m_ref[...] = jnp.full((8, 128), m_tok, jnp.float32)
        pos = pos_ref[...]
        m_pos = jnp.sum(pos) / float(MAX_LEN * EMB)
        qbits = jnp.where(pos > m_pos, jnp.int32(_BF16_ONE), jnp.int32(0))
        qpos_ref[...] = qbits[:, :64] | (qbits[:, 64:] << 16)


def _prep(token_table, pos_table):
    return pl.pallas_call(
        _prep_body,
        grid=(GRID,),
        in_specs=[
            pl.BlockSpec((TOK_BLK, EMB), lambda i: (i, 0)),
            pl.BlockSpec((MAX_LEN, EMB), lambda i: (0, 0)),
        ],
        out_specs=[
            pl.BlockSpec((8, 128), lambda i: (0, 0)),
            pl.BlockSpec((MAX_LEN, EMB // 2), lambda i: (0, 0)),
        ],
        out_shape=[
            jax.ShapeDtypeStruct((8, 128), jnp.float32),
            jax.ShapeDtypeStruct((MAX_LEN, EMB // 2), jnp.int32),
        ],
        scratch_shapes=[pltpu.VMEM((8, 128), jnp.float32)],
    )(token_table, pos_table)


_mesh = plsc.VectorSubcoreMesh(
    core_axis_name="c", subcore_axis_name="s", num_cores=NC, num_subcores=NS
)


@functools.partial(
    pl.kernel,
    out_type=jax.ShapeDtypeStruct((B, T, EMB), jnp.float32),
    mesh=_mesh,
    scratch_types=[
        pltpu.VMEM((2 * BPW, GS0), jnp.int32),
        pltpu.VMEM((T, EMB), jnp.float32),
        pltpu.VMEM((T, EMB), jnp.float32),
        pltpu.VMEM((T, EMB), jnp.float32),
        pltpu.VMEM((MAX_LEN, EMB // 2), jnp.int32),
        pltpu.VMEM((16,), jnp.float32),
        pltpu.SemaphoreType.DMA,
        pltpu.SemaphoreType.DMA,
        pltpu.SemaphoreType.DMA,
        pltpu.SemaphoreType.DMA,
        pltpu.SemaphoreType.DMA,
        pltpu.SemaphoreType.DMA,
    ],
)
def _sc_lookup(seq_hbm, tok_hbm, m_hbm, qpos_hbm, out_hbm,
               idx_all, g0, g1, g2, qpos_v, m_v,
               gsem0, gsem1, gsem2, osem0, osem1, osem2):
    wid = lax.axis_index("s") * NC + lax.axis_index("c")
    base_b = wid * BPW

    pltpu.sync_copy(seq_hbm.at[pl.ds(2 * base_b, 2 * BPW)], idx_all)
    pltpu.sync_copy(qpos_hbm, qpos_v)
    pltpu.sync_copy(m_hbm, m_v)
    vm = m_v[...]

    def gstart(i, g, gsem):
        pltpu.async_copy(
            tok_hbm.at[idx_all.at[2 * i]], g.at[pl.ds(0, GS0)], gsem)
        pltpu.async_copy(
            tok_hbm.at[idx_all.at[2 * i + 1]], g.at[pl.ds(GS1, GS0)], gsem)

    def gwait(g, gsem):
        pltpu.make_async_copy(
            tok_hbm.at[idx_all.at[0]], g.at[pl.ds(0, GS0)], gsem).wait()
        pltpu.make_async_copy(
            tok_hbm.at[idx_all.at[0]], g.at[pl.ds(GS1, GS0)], gsem).wait()

    def ostart(i, o, osem):
        pltpu.async_copy(o, out_hbm.at[base_b + i], osem)

    def owait(o, osem):
        pltpu.make_async_copy(o, out_hbm.at[base_b], osem).wait()

    def compute(g):
        def body_r(r, c):
            for p in range(4):
                xq = qpos_v[r, pl.ds(16 * p, 16)]
                qa = lax.bitcast_convert_type(xq << 16, jnp.float32)
                qb = lax.bitcast_convert_type(xq & jnp.int32(-65536), jnp.float32)
                sa = pl.ds(16 * p, 16)
                sb = pl.ds(64 + 16 * p, 16)
                va = g[r, sa]
                g[r, sa] = jnp.where(va > vm, 1.0, 0.0) + qa
                vb = g[r, sb]
                g[r, sb] = jnp.where(vb > vm, 1.0, 0.0) + qb
            return c

        lax.fori_loop(0, T, body_r, 0)

    slots = ((g0, gsem0, osem0), (g1, gsem1, osem1), (g2, gsem2, osem2))

    def step(i, cur, nxt, prefetch, drain):
        g, gsem, osem = cur
        if prefetch:
            gn, gsemn, osemn = nxt
            if drain:
                owait(gn, osemn)
            gstart(i + 1, gn, gsemn)
        gwait(g, gsem)
        compute(g)
        ostart(i, g, osem)

    gstart(0, g0, gsem0)
    step(0, slots[0], slots[1], True, False)
    step(1, slots[1], slots[2], True, False)

    def group(k, carry):
        c = 3 * k + 2
        step(c + 0, slots[2], slots[0], True, True)
        step(c + 1, slots[0], slots[1], True, True)
        step(c + 2, slots[1], slots[2], True, True)
        return carry

    lax.fori_loop(0, (BPW - 5) // 3, group, 0)
    step(BPW - 3, slots[2], slots[0], True, True)
    step(BPW - 2, slots[0], slots[1], True, True)
    step(BPW - 1, slots[1], slots[2], False, False)
    owait(g2, osem2)
    owait(g0, osem0)
    owait(g1, osem1)


def kernel(seq, token_table, pos_table):
    m8, qpos = _prep(token_table, pos_table)
    mvec = m8[0, :16]
    s = seq.astype(jnp.int32)
    seq3 = jnp.stack([s[:, :GS0], s[:, GS1:]], axis=1).reshape(2 * B, GS0)
    return _sc_lookup(seq3, token_table, mvec, qpos)


# speedup vs baseline: 2.6390x; 1.0383x over previous
import functools

import jax
import jax.numpy as jnp
from jax import lax
from jax.experimental import pallas as pl
from jax.experimental.pallas import tpu as pltpu
from jax.experimental.pallas import tpu_sc as plsc

VOCAB = 100000
MAX_LEN = 200
EMB = 128
B = 1024
T = 200

NC = 2
NS = 16
NW = NC * NS

BPW = B // NW
GS0 = 104
GS1 = T - GS0

TOK_BLK = 10000
GRID = VOCAB // TOK_BLK
SUB = TOK_BLK // 8

_BF16_ONE = 0x3F80


def _prep_body(tok_ref, pos_ref, m_ref, qpos_ref, acc_ref):
    i = pl.program_id(0)

    @pl.when(i == 0)
    def _():
        acc_ref[...] = jnp.zeros((8, 128), jnp.float32)

    acc_ref[...] += jnp.sum(
        tok_ref[...].reshape(SUB, 8, 128), axis=0)

    @pl.when(i == GRID - 1)
    def _():
        m_tok = jnp.sum(acc_ref[...]) / float(VOCAB * EMB)
        m_ref[...] = jnp.full((8, 128), m_tok, jnp.float32)
        pos = pos_ref[...]
        m_pos = jnp.sum(pos) / float(MAX_LEN * EMB)
        qbits = jnp.where(pos > m_pos, jnp.int32(_BF16_ONE), jnp.int32(0))
        qpos_ref[...] = qbits[:, :64] | (qbits[:, 64:] << 16)


def _prep(token_table, pos_table):
    return pl.pallas_call(
        _prep_body,
        grid=(GRID,),
        in_specs=[
            pl.BlockSpec((TOK_BLK, EMB), lambda i: (i, 0)),
            pl.BlockSpec((MAX_LEN, EMB), lambda i: (0, 0)),
        ],
        out_specs=[
            pl.BlockSpec((8, 128), lambda i: (0, 0)),
            pl.BlockSpec((MAX_LEN, EMB // 2), lambda i: (0, 0)),
        ],
        out_shape=[
            jax.ShapeDtypeStruct((8, 128), jnp.float32),
            jax.ShapeDtypeStruct((MAX_LEN, EMB // 2), jnp.int32),
        ],
        scratch_shapes=[pltpu.VMEM((8, 128), jnp.float32)],
    )(token_table, pos_table)


_mesh = plsc.VectorSubcoreMesh(
    core_axis_name="c", subcore_axis_name="s", num_cores=NC, num_subcores=NS
)


@functools.partial(
    pl.kernel,
    out_type=jax.ShapeDtypeStruct((B, T, EMB), jnp.float32),
    mesh=_mesh,
    scratch_types=[
        pltpu.VMEM((2 * BPW, GS0), jnp.int32),
        pltpu.VMEM((T, EMB), jnp.float32),
        pltpu.VMEM((T, EMB), jnp.float32),
        pltpu.VMEM((T, EMB), jnp.float32),
        pltpu.VMEM((MAX_LEN, EMB // 2), jnp.int32),
        pltpu.VMEM((16,), jnp.float32),
        pltpu.SemaphoreType.DMA,
        pltpu.SemaphoreType.DMA,
        pltpu.SemaphoreType.DMA,
        pltpu.SemaphoreType.DMA,
        pltpu.SemaphoreType.DMA,
        pltpu.SemaphoreType.DMA,
    ],
)
def _sc_lookup(seq_hbm, tok_hbm, m_hbm, qpos_hbm, out_hbm,
               idx_all, g0, g1, g2, qpos_v, m_v,
               gsem0, gsem1, gsem2, osem0, osem1, osem2):
    wid = lax.axis_index("s") * NC + lax.axis_index("c")
    base_b = wid * BPW

    pltpu.sync_copy(seq_hbm.at[pl.ds(2 * base_b, 2 * BPW)], idx_all)
    pltpu.sync_copy(qpos_hbm, qpos_v)
    pltpu.sync_copy(m_hbm, m_v)
    vm = m_v[...]

    def gstart(i, g, gsem):
        pltpu.async_copy(
            tok_hbm.at[idx_all.at[2 * i]], g.at[pl.ds(0, GS0)], gsem)
        pltpu.async_copy(
            tok_hbm.at[idx_all.at[2 * i + 1]], g.at[pl.ds(GS1, GS0)], gsem)

    def gwait(g, gsem):
        pltpu.make_async_copy(
            tok_hbm.at[idx_all.at[0]], g.at[pl.ds(0, GS0)], gsem).wait()
        pltpu.make_async_copy(
            tok_hbm.at[idx_all.at[0]], g.at[pl.ds(GS1, GS0)], gsem).wait()

    def ostart(i, o, osem):
        pltpu.async_copy(o, out_hbm.at[base_b + i], osem)

    def owait(o, osem):
        pltpu.make_async_copy(o, out_hbm.at[base_b], osem).wait()

    def compute(g):
        def body_r(r, c):
            for p in range(4):
                xq = qpos_v[r, pl.ds(16 * p, 16)]
                qa = lax.bitcast_convert_type(xq << 16, jnp.float32)
                qb = lax.bitcast_convert_type(xq & jnp.int32(-65536), jnp.float32)
                sa = pl.ds(16 * p, 16)
                sb = pl.ds(64 + 16 * p, 16)
                va = g[r, sa]
                g[r, sa] = jnp.where(va > vm, 1.0, 0.0) + qa
                vb = g[r, sb]
                g[r, sb] = jnp.where(vb > vm, 1.0, 0.0) + qb
            return c

        lax.fori_loop(0, T, body_r, 0)

    slots = ((g0, gsem0, osem0), (g1, gsem1, osem1), (g2, gsem2, osem2))

    def step(i, cur, nxt, prefetch, drain):
        g, gsem, osem = cur
        if prefetch:
            gn, gsemn, osemn = nxt
            if drain:
                owait(gn, osemn)
            gstart(i + 1, gn, gsemn)
        gwait(g, gsem)
        compute(g)
        ostart(i, g, osem)

    gstart(0, g0, gsem0)
    step(0, slots[0], slots[1], True, False)
    step(1, slots[1], slots[2], True, False)

    def group(k, carry):
        c = 3 * k + 2
        step(c + 0, slots[2], slots[0], True, True)
        step(c + 1, slots[0], slots[1], True, True)
        step(c + 2, slots[1], slots[2], True, True)
        return carry

    lax.fori_loop(0, (BPW - 5) // 3, group, 0)
    step(BPW - 3, slots[2], slots[0], True, True)
    step(BPW - 2, slots[0], slots[1], True, True)
    step(BPW - 1, slots[1], slots[2], False, False)
    owait(g2, osem2)
    owait(g0, osem0)
    owait(g1, osem1)


def kernel(seq, token_table, pos_table):
    m8, qpos = _prep(token_table, pos_table)
    mvec = m8[0, :16]
    s = seq.astype(jnp.int32)
    seq3 = jnp.stack([s[:, :GS0], s[:, GS1:]], axis=1).reshape(2 * B, GS0)
    return _sc_lookup(seq3, token_table, mvec, qpos)


# speedup vs baseline: 2.6565x; 1.0066x over previous
import functools

import jax
import jax.numpy as jnp
from jax import lax
from jax.experimental import pallas as pl
from jax.experimental.pallas import tpu as pltpu
from jax.experimental.pallas import tpu_sc as plsc

VOCAB = 100000
MAX_LEN = 200
EMB = 128
B = 1024
T = 200

NC = 2
NS = 16
NW = NC * NS

BPW = B // NW
GS0 = 104
GS1 = T - GS0

TOK_BLK = 25000
GRID = VOCAB // TOK_BLK
SUB = TOK_BLK // 8

_BF16_ONE = 0x3F80


def _prep_body(tok_ref, pos_ref, m_ref, qpos_ref, acc_ref):
    i = pl.program_id(0)

    @pl.when(i == 0)
    def _():
        acc_ref[...] = jnp.zeros((8, 128), jnp.float32)

    acc_ref[...] += jnp.sum(
        tok_ref[...].reshape(SUB, 8, 128), axis=0)

    @pl.when(i == GRID - 1)
    def _():
        m_tok = jnp.sum(acc_ref[...]) / float(VOCAB * EMB)
        m_ref[...] = jnp.full((8, 128), m_tok, jnp.float32)
        pos = pos_ref[...]
        m_pos = jnp.sum(pos) / float(MAX_LEN * EMB)
        qbits = jnp.where(pos > m_pos, jnp.int32(_BF16_ONE), jnp.int32(0))
        qpos_ref[...] = qbits[:, :64] | (qbits[:, 64:] << 16)


def _prep(token_table, pos_table):
    return pl.pallas_call(
        _prep_body,
        grid=(GRID,),
        in_specs=[
            pl.BlockSpec((TOK_BLK, EMB), lambda i: (i, 0)),
            pl.BlockSpec((MAX_LEN, EMB), lambda i: (0, 0)),
        ],
        out_specs=[
            pl.BlockSpec((8, 128), lambda i: (0, 0)),
            pl.BlockSpec((MAX_LEN, EMB // 2), lambda i: (0, 0)),
        ],
        out_shape=[
            jax.ShapeDtypeStruct((8, 128), jnp.float32),
            jax.ShapeDtypeStruct((MAX_LEN, EMB // 2), jnp.int32),
        ],
        scratch_shapes=[pltpu.VMEM((8, 128), jnp.float32)],
    )(token_table, pos_table)


_mesh = plsc.VectorSubcoreMesh(
    core_axis_name="c", subcore_axis_name="s", num_cores=NC, num_subcores=NS
)


@functools.partial(
    pl.kernel,
    out_type=jax.ShapeDtypeStruct((B, T, EMB), jnp.float32),
    mesh=_mesh,
    scratch_types=[
        pltpu.VMEM((2 * BPW, GS0), jnp.int32),
        pltpu.VMEM((T, EMB), jnp.float32),
        pltpu.VMEM((T, EMB), jnp.float32),
        pltpu.VMEM((T, EMB), jnp.float32),
        pltpu.VMEM((MAX_LEN, EMB // 2), jnp.int32),
        pltpu.VMEM((16,), jnp.float32),
        pltpu.SemaphoreType.DMA,
        pltpu.SemaphoreType.DMA,
        pltpu.SemaphoreType.DMA,
        pltpu.SemaphoreType.DMA,
        pltpu.SemaphoreType.DMA,
        pltpu.SemaphoreType.DMA,
    ],
)
def _sc_lookup(seq_hbm, tok_hbm, m_hbm, qpos_hbm, out_hbm,
               idx_all, g0, g1, g2, qpos_v, m_v,
               gsem0, gsem1, gsem2, osem0, osem1, osem2):
    wid = lax.axis_index("s") * NC + lax.axis_index("c")
    base_b = wid * BPW

    pltpu.sync_copy(seq_hbm.at[pl.ds(2 * base_b, 2 * BPW)], idx_all)
    pltpu.sync_copy(qpos_hbm, qpos_v)
    pltpu.sync_copy(m_hbm, m_v)
    vm = m_v[...]

    def gstart(i, g, gsem):
        pltpu.async_copy(
            tok_hbm.at[idx_all.at[2 * i]], g.at[pl.ds(0, GS0)], gsem)
        pltpu.async_copy(
            tok_hbm.at[idx_all.at[2 * i + 1]], g.at[pl.ds(GS1, GS0)], gsem)

    def gwait(g, gsem):
        pltpu.make_async_copy(
            tok_hbm.at[idx_all.at[0]], g.at[pl.ds(0, GS0)], gsem).wait()
        pltpu.make_async_copy(
            tok_hbm.at[idx_all.at[0]], g.at[pl.ds(GS1, GS0)], gsem).wait()

    def ostart(i, o, osem):
        pltpu.async_copy(o, out_hbm.at[base_b + i], osem)

    def owait(o, osem):
        pltpu.make_async_copy(o, out_hbm.at[base_b], osem).wait()

    def compute(g):
        def body_r(r, c):
            for p in range(4):
                xq = qpos_v[r, pl.ds(16 * p, 16)]
                qa = lax.bitcast_convert_type(xq << 16, jnp.float32)
                qb = lax.bitcast_convert_type(xq & jnp.int32(-65536), jnp.float32)
                sa = pl.ds(16 * p, 16)
                sb = pl.ds(64 + 16 * p, 16)
                va = g[r, sa]
                g[r, sa] = jnp.where(va > vm, 1.0, 0.0) + qa
                vb = g[r, sb]
                g[r, sb] = jnp.where(vb > vm, 1.0, 0.0) + qb
            return c

        lax.fori_loop(0, T, body_r, 0)

    slots = ((g0, gsem0, osem0), (g1, gsem1, osem1), (g2, gsem2, osem2))

    def step(i, cur, nxt, prefetch, drain):
        g, gsem, osem = cur
        if prefetch:
            gn, gsemn, osemn = nxt
            if drain:
                owait(gn, osemn)
            gstart(i + 1, gn, gsemn)
        gwait(g, gsem)
        compute(g)
        ostart(i, g, osem)

    gstart(0, g0, gsem0)
    step(0, slots[0], slots[1], True, False)
    step(1, slots[1], slots[2], True, False)

    def group(k, carry):
        c = 3 * k + 2
        step(c + 0, slots[2], slots[0], True, True)
        step(c + 1, slots[0], slots[1], True, True)
        step(c + 2, slots[1], slots[2], True, True)
        return carry

    lax.fori_loop(0, (BPW - 5) // 3, group, 0)
    step(BPW - 3, slots[2], slots[0], True, True)
    step(BPW - 2, slots[0], slots[1], True, True)
    step(BPW - 1, slots[1], slots[2], False, False)
    owait(g2, osem2)
    owait(g0, osem0)
    owait(g1, osem1)


def kernel(seq, token_table, pos_table):
    m8, qpos = _prep(token_table, pos_table)
    mvec = m8[0, :16]
    s = seq.astype(jnp.int32)
    seq3 = jnp.stack([s[:, :GS0], s[:, GS1:]], axis=1).reshape(2 * B, GS0)
    return _sc_lookup(seq3, token_table, mvec, qpos)
